# baseline (speedup 1.0000x reference)
import functools

import jax
import jax.numpy as jnp
import numpy as np
from jax import lax
from jax.experimental import pallas as pl
from jax.experimental.pallas import tpu as pltpu
from jax.experimental.pallas import tpu_sc as plsc

NUM_WORKERS = 32
CHUNKS = (104, 96)

_UNPACK_PERM = np.array(
    [32 * (c // 32) + 2 * (c % 16) + (c % 32) // 16 for c in range(64)],
    dtype=np.int32,
)


def _make_sc_pool(batch, seq, vocab, embed):
    assert batch % NUM_WORKERS == 0
    b_per_w = batch // NUM_WORKERS
    assert sum(CHUNKS) == seq and all(c % 8 == 0 and c <= 128 for c in CHUNKS)
    assert seq % 8 == 0
    idx_per_w = b_per_w * seq
    assert embed % 32 == 0
    npair = embed // 32

    mesh = plsc.VectorSubcoreMesh(core_axis_name="c", subcore_axis_name="s",
                                  num_cores=2, num_subcores=16)

    @functools.partial(
        pl.kernel,
        out_type=jax.ShapeDtypeStruct((batch, embed), jnp.float32),
        mesh=mesh,
        scratch_types=[
            pltpu.VMEM((idx_per_w,), jnp.int32),
            pltpu.VMEM((seq, embed), jnp.bfloat16),
            pltpu.VMEM((seq, embed), jnp.bfloat16),
            pltpu.VMEM((b_per_w, embed), jnp.float32),
            pltpu.SemaphoreType.DMA,
            pltpu.SemaphoreType.DMA,
        ],
        compiler_params=pltpu.CompilerParams(use_tc_tiling_on_sc=False,
                                             needs_layout_passes=False),
    )
    def sc_pool(x_hbm, table_hbm, out_hbm, idx_v, buf_a, buf_b, sout_v,
                sem_a, sem_b):
        wid = lax.axis_index("s") * 2 + lax.axis_index("c")
        base = wid * b_per_w
        bufs = (buf_a, buf_b)
        sems = (sem_a, sem_b)

        pltpu.sync_copy(x_hbm.at[pl.ds(base * seq, idx_per_w)], idx_v)

        def fire(row, buf, sem):
            ibase = row * seq
            off = 0
            for c in CHUNKS:
                pltpu.async_copy(
                    table_hbm.at[idx_v.at[pl.ds(ibase + off, c)]],
                    buf.at[pl.ds(off, c)],
                    sem,
                )
                off += c

        def drain(buf, sem):
            pltpu.make_async_copy(table_hbm.at[pl.ds(0, seq)], buf, sem).wait()

        def accumulate(row, buf):
            def step(t, accs):
                rbase = t * 8
                out = list(accs)
                for k in range(npair):
                    sl = pl.ds(k * 32, 32)
                    ev, od = [], []
                    for r in range(8):
                        a, b = plsc.unpack(buf[rbase + r, sl],
                                           format=plsc.PackFormat.INTERLEAVED)
                        ev.append(a)
                        od.append(b)
                    se = ((ev[0] + ev[1]) + (ev[2] + ev[3])) + \
                         ((ev[4] + ev[5]) + (ev[6] + ev[7]))
                    so = ((od[0] + od[1]) + (od[2] + od[3])) + \
                         ((od[4] + od[5]) + (od[6] + od[7]))
                    out[2 * k] = out[2 * k] + se
                    out[2 * k + 1] = out[2 * k + 1] + so
                return tuple(out)

            zeros = tuple(jnp.zeros((16,), jnp.float32)
                          for _ in range(2 * npair))
            accs = lax.fori_loop(0, seq // 8, step, zeros)
            for k in range(2 * npair):
                sout_v[row, pl.ds(k * 16, 16)] = accs[k]

        fire(0, bufs[0], sems[0])

        @pl.loop(0, b_per_w, step=2)
        def _row_loop(i):
            for b in range(2):
                row = i + b
                nxt = row + 1

                @pl.when(nxt < b_per_w)
                def _():
                    fire(nxt, bufs[1 - b], sems[1 - b])

                drain(bufs[b], sems[b])
                accumulate(row, bufs[b])

        pltpu.sync_copy(sout_v, out_hbm.at[pl.ds(base, b_per_w)])

    return sc_pool


def _mlp_body(ssum_ref, mask_ref, w1_ref, b1_ref, w2_ref, b2_ref, out_ref):
    denom = jnp.maximum(jnp.sum(mask_ref[...], axis=1, keepdims=True), 1.0)
    s = ssum_ref[...] / denom
    h = jnp.dot(s, w1_ref[...], preferred_element_type=jnp.float32)
    h = jnp.maximum(h + b1_ref[...], 0.0)
    out_ref[...] = jnp.dot(h, w2_ref[...],
                           preferred_element_type=jnp.float32) + b2_ref[...]


def kernel(x, mask, table, W1, b1, W2, b2):
    batch, seq = x.shape
    vocab, embed = table.shape
    hidden = W1.shape[1]
    nout = W2.shape[1]

    x_flat = x.astype(jnp.int32).reshape(-1)
    table_bf = table.astype(jnp.bfloat16)
    ssum = _make_sc_pool(batch, seq, vocab, embed)(x_flat, table_bf)
    W1_perm = W1[_UNPACK_PERM, :]

    blk = 512
    grid = (batch // blk,)
    out = pl.pallas_call(
        _mlp_body,
        grid=grid,
        in_specs=[
            pl.BlockSpec((blk, embed), lambda i: (i, 0)),
            pl.BlockSpec((blk, seq), lambda i: (i, 0)),
            pl.BlockSpec((embed, hidden), lambda i: (0, 0)),
            pl.BlockSpec((1, hidden), lambda i: (0, 0)),
            pl.BlockSpec((hidden, nout), lambda i: (0, 0)),
            pl.BlockSpec((1, nout), lambda i: (0, 0)),
        ],
        out_specs=pl.BlockSpec((blk, nout), lambda i: (i, 0)),
        out_shape=jax.ShapeDtypeStruct((batch, nout), jnp.float32),
    )(ssum, mask, W1_perm, b1.reshape(1, -1), W2, b2.reshape(1, -1))
    return out


# speedup vs baseline: 1.8631x; 1.8631x over previous
import functools

import jax
import jax.numpy as jnp
from jax import lax
from jax.experimental import pallas as pl
from jax.experimental.pallas import tpu as pltpu
from jax.experimental.pallas import tpu_sc as plsc

NUM_WORKERS = 32
CHUNKS = (104, 96)
EPAD = 128
TBLK = 8192


def _tpad_body(tt_ref, out_ref):
    out_ref[:, 0:64] = jnp.transpose(tt_ref[...], (1, 0))


def _make_tpad(vocab, embed):
    return pl.pallas_call(
        _tpad_body,
        grid=(pl.cdiv(vocab, TBLK),),
        in_specs=[pl.BlockSpec((embed, TBLK), lambda i: (0, i))],
        out_specs=pl.BlockSpec((TBLK, EPAD), lambda i: (i, 0)),
        out_shape=jax.ShapeDtypeStruct((vocab, EPAD), jnp.float32),
        compiler_params=pltpu.CompilerParams(
            dimension_semantics=("arbitrary",)),
    )


def _make_sc_pool(batch, seq, vocab, embed):
    assert batch % NUM_WORKERS == 0
    b_per_w = batch // NUM_WORKERS
    assert sum(CHUNKS) == seq and all(c % 8 == 0 and c <= 128 for c in CHUNKS)
    assert seq % 8 == 0
    idx_per_w = b_per_w * seq
    assert embed % 16 == 0
    nvec = embed // 16

    mesh = plsc.VectorSubcoreMesh(core_axis_name="c", subcore_axis_name="s",
                                  num_cores=2, num_subcores=16)

    @functools.partial(
        pl.kernel,
        out_type=jax.ShapeDtypeStruct((batch, embed), jnp.float32),
        mesh=mesh,
        scratch_types=[
            pltpu.VMEM((idx_per_w,), jnp.int32),
            pltpu.VMEM((seq, EPAD), jnp.float32),
            pltpu.VMEM((seq, EPAD), jnp.float32),
            pltpu.VMEM((b_per_w, embed), jnp.float32),
            pltpu.SemaphoreType.DMA,
            pltpu.SemaphoreType.DMA,
        ],
        compiler_params=pltpu.CompilerParams(use_tc_tiling_on_sc=True),
    )
    def sc_pool(x_hbm, table_hbm, out_hbm, idx_v, buf_a, buf_b, sout_v,
                sem_a, sem_b):
        wid = lax.axis_index("s") * 2 + lax.axis_index("c")
        base = wid * b_per_w
        bufs = (buf_a, buf_b)
        sems = (sem_a, sem_b)

        pltpu.sync_copy(x_hbm.at[pl.ds(base * seq, idx_per_w)], idx_v)

        def fire(row, buf, sem):
            ibase = row * seq
            off = 0
            for c in CHUNKS:
                pltpu.async_copy(
                    table_hbm.at[idx_v.at[pl.ds(ibase + off, c)]],
                    buf.at[pl.ds(off, c)],
                    sem,
                )
                off += c

        def drain(buf, sem):
            pltpu.make_async_copy(table_hbm.at[pl.ds(0, seq)], buf, sem).wait()

        def accumulate(row, buf):
            def step(t, accs):
                rbase = t * 8
                out = []
                for k in range(nvec):
                    sl = pl.ds(k * 16, 16)
                    l = [buf[rbase + r, sl] for r in range(8)]
                    s = ((l[0] + l[1]) + (l[2] + l[3])) + \
                        ((l[4] + l[5]) + (l[6] + l[7]))
                    out.append(accs[k] + s)
                return tuple(out)

            zeros = tuple(jnp.zeros((16,), jnp.float32) for _ in range(nvec))
            accs = lax.fori_loop(0, seq // 8, step, zeros)
            for k in range(nvec):
                sout_v[row, pl.ds(k * 16, 16)] = accs[k]

        fire(0, bufs[0], sems[0])

        @pl.loop(0, b_per_w, step=2)
        def _row_loop(i):
            for b in range(2):
                row = i + b
                nxt = row + 1

                @pl.when(nxt < b_per_w)
                def _():
                    fire(nxt, bufs[1 - b], sems[1 - b])

                drain(bufs[b], sems[b])
                accumulate(row, bufs[b])

        pltpu.sync_copy(sout_v, out_hbm.at[pl.ds(base, b_per_w)])

    return sc_pool


def _mlp_body(ssum_ref, mask_ref, w1_ref, b1_ref, w2_ref, b2_ref, out_ref):
    denom = jnp.maximum(jnp.sum(mask_ref[...], axis=1, keepdims=True), 1.0)
    s = ssum_ref[...] / denom
    h = jnp.dot(s, w1_ref[...], preferred_element_type=jnp.float32)
    h = jnp.maximum(h + b1_ref[...], 0.0)
    out_ref[...] = jnp.dot(h, w2_ref[...],
                           preferred_element_type=jnp.float32) + b2_ref[...]


def kernel(x, mask, table, W1, b1, W2, b2):
    batch, seq = x.shape
    vocab, embed = table.shape
    hidden = W1.shape[1]
    nout = W2.shape[1]

    x_flat = x.astype(jnp.int32).reshape(-1)
    table_pad = _make_tpad(vocab, embed)(table.T)
    ssum = _make_sc_pool(batch, seq, vocab, embed)(x_flat, table_pad)

    blk = 512
    grid = (batch // blk,)
    out = pl.pallas_call(
        _mlp_body,
        grid=grid,
        in_specs=[
            pl.BlockSpec((blk, embed), lambda i: (i, 0)),
            pl.BlockSpec((blk, seq), lambda i: (i, 0)),
            pl.BlockSpec((embed, hidden), lambda i: (0, 0)),
            pl.BlockSpec((1, hidden), lambda i: (0, 0)),
            pl.BlockSpec((hidden, nout), lambda i: (0, 0)),
            pl.BlockSpec((1, nout), lambda i: (0, 0)),
        ],
        out_specs=pl.BlockSpec((blk, nout), lambda i: (i, 0)),
        out_shape=jax.ShapeDtypeStruct((batch, nout), jnp.float32),
    )(ssum, mask, W1, b1.reshape(1, -1), W2, b2.reshape(1, -1))
    return out


# speedup vs baseline: 1.9468x; 1.0449x over previous
import functools

import jax
import jax.numpy as jnp
from jax import lax
from jax.experimental import pallas as pl
from jax.experimental.pallas import tpu as pltpu
from jax.experimental.pallas import tpu_sc as plsc

NUM_WORKERS = 32
CHUNKS = (104, 96)
EPAD = 128
TBLK = 8192


def _tpad_body(tt_ref, out_ref):
    out_ref[:, 0:64] = jnp.transpose(tt_ref[...], (1, 0))


def _make_tpad(vocab, embed):
    return pl.pallas_call(
        _tpad_body,
        grid=(pl.cdiv(vocab, TBLK),),
        in_specs=[pl.BlockSpec((embed, TBLK), lambda i: (0, i))],
        out_specs=pl.BlockSpec((TBLK, EPAD), lambda i: (i, 0)),
        out_shape=jax.ShapeDtypeStruct((vocab, EPAD), jnp.float32),
        compiler_params=pltpu.CompilerParams(
            dimension_semantics=("arbitrary",)),
    )


def _make_sc_pool(batch, seq, vocab, embed):
    assert batch % NUM_WORKERS == 0
    b_per_w = batch // NUM_WORKERS
    assert sum(CHUNKS) == seq and all(c % 8 == 0 and c <= 128 for c in CHUNKS)
    assert seq % 8 == 0
    idx_per_w = b_per_w * seq
    assert embed % 16 == 0
    nvec = embed // 16

    mesh = plsc.VectorSubcoreMesh(core_axis_name="c", subcore_axis_name="s",
                                  num_cores=2, num_subcores=16)

    @functools.partial(
        pl.kernel,
        out_type=jax.ShapeDtypeStruct((batch, embed), jnp.float32),
        mesh=mesh,
        scratch_types=[
            pltpu.VMEM((idx_per_w,), jnp.int32),
            pltpu.VMEM((seq, EPAD), jnp.float32),
            pltpu.VMEM((seq, EPAD), jnp.float32),
            pltpu.VMEM((seq, EPAD), jnp.float32),
            pltpu.VMEM((b_per_w, embed), jnp.float32),
            pltpu.SemaphoreType.DMA,
            pltpu.SemaphoreType.DMA,
            pltpu.SemaphoreType.DMA,
        ],
        compiler_params=pltpu.CompilerParams(use_tc_tiling_on_sc=True),
    )
    def sc_pool(x_hbm, table_hbm, out_hbm, idx_v, buf_a, buf_b, buf_c,
                sout_v, sem_a, sem_b, sem_c):
        wid = lax.axis_index("s") * 2 + lax.axis_index("c")
        base = wid * b_per_w
        bufs = (buf_a, buf_b, buf_c)
        sems = (sem_a, sem_b, sem_c)

        pltpu.sync_copy(x_hbm.at[pl.ds(base * seq, idx_per_w)], idx_v)

        def fire(row, buf, sem):
            ibase = row * seq
            off = 0
            for c in CHUNKS:
                pltpu.async_copy(
                    table_hbm.at[idx_v.at[pl.ds(ibase + off, c)]],
                    buf.at[pl.ds(off, c)],
                    sem,
                )
                off += c

        def drain(buf, sem):
            pltpu.make_async_copy(table_hbm.at[pl.ds(0, seq)], buf, sem).wait()

        def accumulate(row, buf):
            def step(t, accs):
                rbase = t * 8
                out = []
                for k in range(nvec):
                    sl = pl.ds(k * 16, 16)
                    l = [buf[rbase + r, sl] for r in range(8)]
                    s = ((l[0] + l[1]) + (l[2] + l[3])) + \
                        ((l[4] + l[5]) + (l[6] + l[7]))
                    out.append(accs[k] + s)
                return tuple(out)

            zeros = tuple(jnp.zeros((16,), jnp.float32) for _ in range(nvec))
            accs = lax.fori_loop(0, seq // 8, step, zeros)
            for k in range(nvec):
                sout_v[row, pl.ds(k * 16, 16)] = accs[k]

        fire(0, bufs[0], sems[0])
        fire(1, bufs[1], sems[1])

        @pl.loop(0, b_per_w, step=3)
        def _row_loop(i):
            for b in range(3):
                row = i + b
                nxt = row + 2
                fslot = (b + 2) % 3

                @pl.when(nxt < b_per_w)
                def _():
                    fire(nxt, bufs[fslot], sems[fslot])

                @pl.when(row < b_per_w)
                def _():
                    drain(bufs[b], sems[b])
                    accumulate(row, bufs[b])

        pltpu.sync_copy(sout_v, out_hbm.at[pl.ds(base, b_per_w)])

    return sc_pool


def _mlp_body(ssum_ref, mask_ref, w1_ref, b1_ref, w2_ref, b2_ref, out_ref):
    denom = jnp.maximum(jnp.sum(mask_ref[...], axis=1, keepdims=True), 1.0)
    s = ssum_ref[...] / denom
    h = jnp.dot(s, w1_ref[...], preferred_element_type=jnp.float32)
    h = jnp.maximum(h + b1_ref[...], 0.0)
    out_ref[...] = jnp.dot(h, w2_ref[...],
                           preferred_element_type=jnp.float32) + b2_ref[...]


def kernel(x, mask, table, W1, b1, W2, b2):
    batch, seq = x.shape
    vocab, embed = table.shape
    hidden = W1.shape[1]
    nout = W2.shape[1]

    x_flat = x.astype(jnp.int32).reshape(-1)
    table_pad = _make_tpad(vocab, embed)(table.T)
    ssum = _make_sc_pool(batch, seq, vocab, embed)(x_flat, table_pad)

    blk = 512
    grid = (batch // blk,)
    out = pl.pallas_call(
        _mlp_body,
        grid=grid,
        in_specs=[
            pl.BlockSpec((blk, embed), lambda i: (i, 0)),
            pl.BlockSpec((blk, seq), lambda i: (i, 0)),
            pl.BlockSpec((embed, hidden), lambda i: (0, 0)),
            pl.BlockSpec((1, hidden), lambda i: (0, 0)),
            pl.BlockSpec((hidden, nout), lambda i: (0, 0)),
            pl.BlockSpec((1, nout), lambda i: (0, 0)),
        ],
        out_specs=pl.BlockSpec((blk, nout), lambda i: (i, 0)),
        out_shape=jax.ShapeDtypeStruct((batch, nout), jnp.float32),
    )(ssum, mask, W1, b1.reshape(1, -1), W2, b2.reshape(1, -1))
    return out


# speedup vs baseline: 2.3849x; 1.2250x over previous
import functools

import jax
import jax.numpy as jnp
from jax import lax
from jax.experimental import pallas as pl
from jax.experimental.pallas import tpu as pltpu
from jax.experimental.pallas import tpu_sc as plsc

NUM_WORKERS = 32
CHUNKS = (104, 96)
EPAD = 128
TBLK = 7680


GRP = 320


def _tpad_body(tt_ref, out_ref):
    t = jnp.transpose(tt_ref[...], (1, 0))
    g = t.reshape(TBLK // GRP, 2, GRP // 2, 64)
    out_ref[:, 0:64] = g[:, 0].reshape(TBLK // 2, 64)
    out_ref[:, 64:128] = g[:, 1].reshape(TBLK // 2, 64)


def _make_tpad(vocab, embed):
    assert TBLK % GRP == 0 and (GRP // 2) % 8 == 0 and vocab % GRP == 0
    return pl.pallas_call(
        _tpad_body,
        grid=(pl.cdiv(vocab, TBLK),),
        in_specs=[pl.BlockSpec((embed, TBLK), lambda i: (0, i))],
        out_specs=pl.BlockSpec((TBLK // 2, 128), lambda i: (i, 0)),
        out_shape=jax.ShapeDtypeStruct((vocab // 2, 128), jnp.float32),
        compiler_params=pltpu.CompilerParams(
            dimension_semantics=("arbitrary",)),
    )


def _make_sc_pool(batch, seq, vocab, embed):
    assert batch % NUM_WORKERS == 0
    b_per_w = batch // NUM_WORKERS
    assert sum(CHUNKS) == seq and all(c % 8 == 0 and c <= 128 for c in CHUNKS)
    assert seq % 8 == 0
    idx_per_w = b_per_w * seq
    assert embed % 16 == 0
    nvec = embed // 16

    mesh = plsc.VectorSubcoreMesh(core_axis_name="c", subcore_axis_name="s",
                                  num_cores=2, num_subcores=16)

    @functools.partial(
        pl.kernel,
        out_type=jax.ShapeDtypeStruct((batch, embed), jnp.float32),
        mesh=mesh,
        scratch_types=[
            pltpu.VMEM((idx_per_w,), jnp.int32),
            pltpu.VMEM((seq, embed), jnp.float32),
            pltpu.VMEM((seq, embed), jnp.float32),
            pltpu.VMEM((seq, embed), jnp.float32),
            pltpu.VMEM((b_per_w, embed), jnp.float32),
            pltpu.SemaphoreType.DMA,
            pltpu.SemaphoreType.DMA,
            pltpu.SemaphoreType.DMA,
        ],
        compiler_params=pltpu.CompilerParams(use_tc_tiling_on_sc=False),
    )
    def sc_pool(x_hbm, table_hbm, out_hbm, idx_v, buf_a, buf_b, buf_c,
                sout_v, sem_a, sem_b, sem_c):
        wid = lax.axis_index("s") * 2 + lax.axis_index("c")
        base = wid * b_per_w
        bufs = (buf_a, buf_b, buf_c)
        sems = (sem_a, sem_b, sem_c)

        pltpu.sync_copy(x_hbm.at[pl.ds(base * seq, idx_per_w)], idx_v)

        def fire(row, buf, sem):
            ibase = row * seq
            off = 0
            for c in CHUNKS:
                pltpu.async_copy(
                    table_hbm.at[idx_v.at[pl.ds(ibase + off, c)]],
                    buf.at[pl.ds(off, c)],
                    sem,
                )
                off += c

        def drain(buf, sem):
            pltpu.make_async_copy(table_hbm.at[pl.ds(0, seq)], buf, sem).wait()

        def accumulate(row, buf):
            def step(t, accs):
                rbase = t * 8
                out = []
                for k in range(nvec):
                    sl = pl.ds(k * 16, 16)
                    l = [buf[rbase + r, sl] for r in range(8)]
                    s = ((l[0] + l[1]) + (l[2] + l[3])) + \
                        ((l[4] + l[5]) + (l[6] + l[7]))
                    out.append(accs[k] + s)
                return tuple(out)

            zeros = tuple(jnp.zeros((16,), jnp.float32) for _ in range(nvec))
            accs = lax.fori_loop(0, seq // 8, step, zeros)
            for k in range(nvec):
                sout_v[row, pl.ds(k * 16, 16)] = accs[k]

        fire(0, bufs[0], sems[0])
        fire(1, bufs[1], sems[1])

        @pl.loop(0, b_per_w, step=3)
        def _row_loop(i):
            for b in range(3):
                row = i + b
                nxt = row + 2
                fslot = (b + 2) % 3

                @pl.when(nxt < b_per_w)
                def _():
                    fire(nxt, bufs[fslot], sems[fslot])

                @pl.when(row < b_per_w)
                def _():
                    drain(bufs[b], sems[b])
                    accumulate(row, bufs[b])

        pltpu.sync_copy(sout_v, out_hbm.at[pl.ds(base, b_per_w)])

    return sc_pool


def _mlp_body(ssum_ref, mask_ref, w1_ref, b1_ref, w2_ref, b2_ref, out_ref):
    denom = jnp.maximum(jnp.sum(mask_ref[...], axis=1, keepdims=True), 1.0)
    s = ssum_ref[...] / denom
    h = jnp.dot(s, w1_ref[...], preferred_element_type=jnp.float32)
    h = jnp.maximum(h + b1_ref[...], 0.0)
    out_ref[...] = jnp.dot(h, w2_ref[...],
                           preferred_element_type=jnp.float32) + b2_ref[...]


def kernel(x, mask, table, W1, b1, W2, b2):
    batch, seq = x.shape
    vocab, embed = table.shape
    hidden = W1.shape[1]
    nout = W2.shape[1]

    x32 = x.astype(jnp.int32).reshape(-1)
    r = x32 % GRP
    x_flat = x32 - r + (r % (GRP // 2)) * 2 + r // (GRP // 2)
    table_pack = _make_tpad(vocab, embed)(table.T)
    table_lin = table_pack.reshape(vocab, embed)
    ssum = _make_sc_pool(batch, seq, vocab, embed)(x_flat, table_lin)

    blk = 512
    grid = (batch // blk,)
    out = pl.pallas_call(
        _mlp_body,
        grid=grid,
        in_specs=[
            pl.BlockSpec((blk, embed), lambda i: (i, 0)),
            pl.BlockSpec((blk, seq), lambda i: (i, 0)),
            pl.BlockSpec((embed, hidden), lambda i: (0, 0)),
            pl.BlockSpec((1, hidden), lambda i: (0, 0)),
            pl.BlockSpec((hidden, nout), lambda i: (0, 0)),
            pl.BlockSpec((1, nout), lambda i: (0, 0)),
        ],
        out_specs=pl.BlockSpec((blk, nout), lambda i: (i, 0)),
        out_shape=jax.ShapeDtypeStruct((batch, nout), jnp.float32),
    )(ssum, mask, W1, b1.reshape(1, -1), W2, b2.reshape(1, -1))
    return out


# speedup vs baseline: 2.7498x; 1.1530x over previous
import functools

import jax
import jax.numpy as jnp
from jax import lax
from jax.experimental import pallas as pl
from jax.experimental.pallas import tpu as pltpu
from jax.experimental.pallas import tpu_sc as plsc

NUM_WORKERS = 32
CHUNKS = (104, 96)
EPAD = 128
TBLK = 30720


GRP = 320


def _tpad_body(tt_ref, out_ref):
    t = jnp.transpose(tt_ref[...], (1, 0))
    g = t.reshape(TBLK // GRP, 2, GRP // 2, 64)
    out_ref[:, 0:64] = g[:, 0].reshape(TBLK // 2, 64)
    out_ref[:, 64:128] = g[:, 1].reshape(TBLK // 2, 64)


def _make_tpad(vocab, embed):
    assert TBLK % GRP == 0 and (GRP // 2) % 8 == 0 and vocab % GRP == 0
    return pl.pallas_call(
        _tpad_body,
        grid=(pl.cdiv(vocab, TBLK),),
        in_specs=[pl.BlockSpec((embed, TBLK), lambda i: (0, i))],
        out_specs=pl.BlockSpec((TBLK // 2, 128), lambda i: (i, 0)),
        out_shape=jax.ShapeDtypeStruct((vocab // 2, 128), jnp.float32),
        compiler_params=pltpu.CompilerParams(
            dimension_semantics=("arbitrary",)),
    )


def _make_sc_pool(batch, seq, vocab, embed):
    assert batch % NUM_WORKERS == 0
    b_per_w = batch // NUM_WORKERS
    assert sum(CHUNKS) == seq and all(c % 8 == 0 and c <= 128 for c in CHUNKS)
    assert seq % 8 == 0
    idx_per_w = b_per_w * seq
    assert embed % 16 == 0
    nvec = embed // 16

    mesh = plsc.VectorSubcoreMesh(core_axis_name="c", subcore_axis_name="s",
                                  num_cores=2, num_subcores=16)

    @functools.partial(
        pl.kernel,
        out_type=jax.ShapeDtypeStruct((batch, embed), jnp.float32),
        mesh=mesh,
        scratch_types=[
            pltpu.VMEM((idx_per_w,), jnp.int32),
            pltpu.VMEM((seq, embed), jnp.float32),
            pltpu.VMEM((seq, embed), jnp.float32),
            pltpu.VMEM((seq, embed), jnp.float32),
            pltpu.VMEM((b_per_w, embed), jnp.float32),
            pltpu.SemaphoreType.DMA,
            pltpu.SemaphoreType.DMA,
            pltpu.SemaphoreType.DMA,
        ],
        compiler_params=pltpu.CompilerParams(use_tc_tiling_on_sc=False),
    )
    def sc_pool(x_hbm, table_hbm, out_hbm, idx_v, buf_a, buf_b, buf_c,
                sout_v, sem_a, sem_b, sem_c):
        wid = lax.axis_index("s") * 2 + lax.axis_index("c")
        base = wid * b_per_w
        bufs = (buf_a, buf_b, buf_c)
        sems = (sem_a, sem_b, sem_c)

        pltpu.sync_copy(x_hbm.at[pl.ds(base * seq, idx_per_w)], idx_v)

        def fire(row, buf, sem):
            ibase = row * seq
            off = 0
            for c in CHUNKS:
                pltpu.async_copy(
                    table_hbm.at[idx_v.at[pl.ds(ibase + off, c)]],
                    buf.at[pl.ds(off, c)],
                    sem,
                )
                off += c

        def drain(buf, sem):
            pltpu.make_async_copy(table_hbm.at[pl.ds(0, seq)], buf, sem).wait()

        def accumulate(row, buf):
            def step(t, accs):
                rbase = t * 8
                out = []
                for k in range(nvec):
                    sl = pl.ds(k * 16, 16)
                    l = [buf[rbase + r, sl] for r in range(8)]
                    s = ((l[0] + l[1]) + (l[2] + l[3])) + \
                        ((l[4] + l[5]) + (l[6] + l[7]))
                    out.append(accs[k] + s)
                return tuple(out)

            zeros = tuple(jnp.zeros((16,), jnp.float32) for _ in range(nvec))
            accs = lax.fori_loop(0, seq // 8, step, zeros)
            for k in range(nvec):
                sout_v[row, pl.ds(k * 16, 16)] = accs[k]

        fire(0, bufs[0], sems[0])
        fire(1, bufs[1], sems[1])

        @pl.loop(0, b_per_w, step=3)
        def _row_loop(i):
            for b in range(3):
                row = i + b
                nxt = row + 2
                fslot = (b + 2) % 3

                @pl.when(nxt < b_per_w)
                def _():
                    fire(nxt, bufs[fslot], sems[fslot])

                @pl.when(row < b_per_w)
                def _():
                    drain(bufs[b], sems[b])
                    accumulate(row, bufs[b])

        pltpu.sync_copy(sout_v, out_hbm.at[pl.ds(base, b_per_w)])

    return sc_pool


def _mlp_body(ssum_ref, mask_ref, w1_ref, b1_ref, w2_ref, b2_ref, out_ref):
    denom = jnp.maximum(jnp.sum(mask_ref[...], axis=1, keepdims=True), 1.0)
    s = ssum_ref[...] / denom
    h = jnp.dot(s, w1_ref[...], preferred_element_type=jnp.float32)
    h = jnp.maximum(h + b1_ref[...], 0.0)
    out_ref[...] = jnp.dot(h, w2_ref[...],
                           preferred_element_type=jnp.float32) + b2_ref[...]


def kernel(x, mask, table, W1, b1, W2, b2):
    batch, seq = x.shape
    vocab, embed = table.shape
    hidden = W1.shape[1]
    nout = W2.shape[1]

    x32 = x.astype(jnp.int32).reshape(-1)
    r = x32 % GRP
    x_flat = x32 - r + (r % (GRP // 2)) * 2 + r // (GRP // 2)
    table_pack = _make_tpad(vocab, embed)(table.T)
    table_lin = table_pack.reshape(vocab, embed)
    ssum = _make_sc_pool(batch, seq, vocab, embed)(x_flat, table_lin)

    blk = 512
    grid = (batch // blk,)
    out = pl.pallas_call(
        _mlp_body,
        grid=grid,
        in_specs=[
            pl.BlockSpec((blk, embed), lambda i: (i, 0)),
            pl.BlockSpec((blk, seq), lambda i: (i, 0)),
            pl.BlockSpec((embed, hidden), lambda i: (0, 0)),
            pl.BlockSpec((1, hidden), lambda i: (0, 0)),
            pl.BlockSpec((hidden, nout), lambda i: (0, 0)),
            pl.BlockSpec((1, nout), lambda i: (0, 0)),
        ],
        out_specs=pl.BlockSpec((blk, nout), lambda i: (i, 0)),
        out_shape=jax.ShapeDtypeStruct((batch, nout), jnp.float32),
    )(ssum, mask, W1, b1.reshape(1, -1), W2, b2.reshape(1, -1))
    return out


# speedup vs baseline: 3.4327x; 1.2483x over previous
import functools

import jax
import jax.numpy as jnp
from jax import lax
from jax.experimental import pallas as pl
from jax.experimental.pallas import tpu as pltpu
from jax.experimental.pallas import tpu_sc as plsc

NUM_WORKERS = 32
CHUNKS = (104, 96)
EPAD = 128
TBLK = 46080


GRP = 320


def _tpad_body(tt_ref, out_ref):
    t = jnp.transpose(tt_ref[...], (1, 0))
    g = t.reshape(TBLK // GRP, 2, GRP // 2, 64)
    out_ref[:, 0:64] = g[:, 0].reshape(TBLK // 2, 64)
    out_ref[:, 64:128] = g[:, 1].reshape(TBLK // 2, 64)


def _make_tpad(vocab, embed):
    assert TBLK % GRP == 0 and (GRP // 2) % 8 == 0 and vocab % GRP == 0
    return pl.pallas_call(
        _tpad_body,
        grid=(pl.cdiv(vocab, TBLK),),
        in_specs=[pl.BlockSpec((embed, TBLK), lambda i: (0, i))],
        out_specs=pl.BlockSpec((TBLK // 2, 128), lambda i: (i, 0)),
        out_shape=jax.ShapeDtypeStruct((vocab // 2, 128), jnp.float32),
        compiler_params=pltpu.CompilerParams(
            dimension_semantics=("arbitrary",)),
    )


def _make_sc_pool(batch, seq, vocab, embed):
    assert batch % NUM_WORKERS == 0
    b_per_w = batch // NUM_WORKERS
    assert sum(CHUNKS) == seq and all(c % 8 == 0 and c <= 128 for c in CHUNKS)
    assert seq % 8 == 0
    idx_per_w = b_per_w * seq
    assert embed % 16 == 0
    nvec = embed // 16

    mesh = plsc.VectorSubcoreMesh(core_axis_name="c", subcore_axis_name="s",
                                  num_cores=2, num_subcores=16)

    @functools.partial(
        pl.kernel,
        out_type=jax.ShapeDtypeStruct((batch, embed), jnp.float32),
        mesh=mesh,
        scratch_types=[
            pltpu.VMEM((idx_per_w,), jnp.int32),
            pltpu.VMEM((seq, embed), jnp.float32),
            pltpu.VMEM((seq, embed), jnp.float32),
            pltpu.VMEM((seq, embed), jnp.float32),
            pltpu.VMEM((b_per_w, embed), jnp.float32),
            pltpu.SemaphoreType.DMA,
            pltpu.SemaphoreType.DMA,
            pltpu.SemaphoreType.DMA,
        ],
        compiler_params=pltpu.CompilerParams(use_tc_tiling_on_sc=False),
    )
    def sc_pool(x_hbm, table_hbm, out_hbm, idx_v, buf_a, buf_b, buf_c,
                sout_v, sem_a, sem_b, sem_c):
        wid = lax.axis_index("s") * 2 + lax.axis_index("c")
        base = wid * b_per_w
        bufs = (buf_a, buf_b, buf_c)
        sems = (sem_a, sem_b, sem_c)

        pltpu.sync_copy(x_hbm.at[pl.ds(base * seq, idx_per_w)], idx_v)

        def fire(row, buf, sem):
            ibase = row * seq
            off = 0
            for c in CHUNKS:
                pltpu.async_copy(
                    table_hbm.at[idx_v.at[pl.ds(ibase + off, c)]],
                    buf.at[pl.ds(off, c)],
                    sem,
                )
                off += c

        def drain(buf, sem):
            pltpu.make_async_copy(table_hbm.at[pl.ds(0, seq)], buf, sem).wait()

        def accumulate(row, buf):
            def step(t, accs):
                rbase = t * 8
                out = []
                for k in range(nvec):
                    sl = pl.ds(k * 16, 16)
                    l = [buf[rbase + r, sl] for r in range(8)]
                    s = ((l[0] + l[1]) + (l[2] + l[3])) + \
                        ((l[4] + l[5]) + (l[6] + l[7]))
                    out.append(accs[k] + s)
                return tuple(out)

            zeros = tuple(jnp.zeros((16,), jnp.float32) for _ in range(nvec))
            accs = lax.fori_loop(0, seq // 8, step, zeros)
            for k in range(nvec):
                sout_v[row, pl.ds(k * 16, 16)] = accs[k]

        fire(0, bufs[0], sems[0])
        fire(1, bufs[1], sems[1])

        @pl.loop(0, b_per_w, step=3)
        def _row_loop(i):
            for b in range(3):
                row = i + b
                nxt = row + 2
                fslot = (b + 2) % 3

                @pl.when(nxt < b_per_w)
                def _():
                    fire(nxt, bufs[fslot], sems[fslot])

                @pl.when(row < b_per_w)
                def _():
                    drain(bufs[b], sems[b])
                    accumulate(row, bufs[b])

        pltpu.sync_copy(sout_v, out_hbm.at[pl.ds(base, b_per_w)])

    return sc_pool


def _mlp_body(ssum_ref, mask_ref, w1_ref, b1_ref, w2_ref, b2_ref, out_ref):
    denom = jnp.maximum(jnp.sum(mask_ref[...], axis=1, keepdims=True), 1.0)
    s = ssum_ref[...] / denom
    h = jnp.dot(s, w1_ref[...], preferred_element_type=jnp.float32)
    h = jnp.maximum(h + b1_ref[...], 0.0)
    out_ref[...] = jnp.dot(h, w2_ref[...],
                           preferred_element_type=jnp.float32) + b2_ref[...]


def kernel(x, mask, table, W1, b1, W2, b2):
    batch, seq = x.shape
    vocab, embed = table.shape
    hidden = W1.shape[1]
    nout = W2.shape[1]

    x32 = x.astype(jnp.int32).reshape(-1)
    r = x32 % GRP
    x_flat = x32 - r + (r % (GRP // 2)) * 2 + r // (GRP // 2)
    table_pack = _make_tpad(vocab, embed)(table.T)
    table_lin = table_pack.reshape(vocab, embed)
    ssum = _make_sc_pool(batch, seq, vocab, embed)(x_flat, table_lin)

    blk = 512
    grid = (batch // blk,)
    out = pl.pallas_call(
        _mlp_body,
        grid=grid,
        in_specs=[
            pl.BlockSpec((blk, embed), lambda i: (i, 0)),
            pl.BlockSpec((blk, seq), lambda i: (i, 0)),
            pl.BlockSpec((embed, hidden), lambda i: (0, 0)),
            pl.BlockSpec((1, hidden), lambda i: (0, 0)),
            pl.BlockSpec((hidden, nout), lambda i: (0, 0)),
            pl.BlockSpec((1, nout), lambda i: (0, 0)),
        ],
        out_specs=pl.BlockSpec((blk, nout), lambda i: (i, 0)),
        out_shape=jax.ShapeDtypeStruct((batch, nout), jnp.float32),
    )(ssum, mask, W1, b1.reshape(1, -1), W2, b2.reshape(1, -1))
    return out
